# baseline (speedup 1.0000x reference)
import functools

import jax
import jax.numpy as jnp
from jax import lax
from jax.experimental import pallas as pl
from jax.experimental.pallas import tpu as pltpu
from jax.experimental.pallas import tpu_sc as plsc

N_NODES = 10000
N_EDGES = 160000
IN_CH = 256
OUT_CH = 256
BATCH = 2

_H = 128
_NS = 16
_NC = 2
_CHUNK = 64
_CPT = 160
_EPAD = _NS * _CPT * _CHUNK
_NP = 10240
_RPT = _NP // _NS
_R = 1000
_NB = N_NODES // _R


def _spmm_tables(tab, src_g, dst_p, val_p, zrows):
    mesh = plsc.VectorSubcoreMesh(
        core_axis_name="c", subcore_axis_name="s",
        num_cores=_NC, num_subcores=_NS)

    @functools.partial(
        pl.kernel,
        out_type=jax.ShapeDtypeStruct((2 * BATCH * _NP, _H), jnp.float32),
        mesh=mesh,
        scratch_types=[
            pltpu.VMEM_SHARED((_NP, _H), jnp.float32),
            [pltpu.VMEM((_CHUNK,), jnp.int32) for _ in range(4)],
            [pltpu.VMEM((_CHUNK,), jnp.int32) for _ in range(4)],
            [pltpu.VMEM((_CHUNK,), jnp.float32) for _ in range(4)],
            [pltpu.VMEM((_CHUNK, _H), jnp.float32) for _ in range(4)],
            [pltpu.SemaphoreType.DMA for _ in range(4)],
            [pltpu.SemaphoreType.DMA for _ in range(4)],
            [pltpu.SemaphoreType.DMA for _ in range(4)],
            [pltpu.SemaphoreType.DMA for _ in range(4)],
            [pltpu.SemaphoreType.DMA for _ in range(4)],
        ],
    )
    def k(tab_h, srcg_h, dstp_h, valp_h, zrows_h, out_h,
          acc, srcv, dstv, valv, rows, isem, dsem, vsem, gsem, ssem):
        c = lax.axis_index("c")
        s = lax.axis_index("s")

        def zero_acc():
            pltpu.sync_copy(zrows_h, acc.at[pl.ds(s * _RPT, _RPT)])

        zero_acc()
        plsc.subcore_barrier()

        def scale(j, a):
            rp = rows[a]

            def grp(g, carry):
                v16 = valv[a][pl.ds(g * 16, 16)]
                for l in range(16):
                    v = v16[l]
                    e = g * 16 + l
                    for q in range(_H // 16):
                        sl = pl.ds(q * 16, 16)
                        rp[e, sl] = rp[e, sl] * v
                return carry
            lax.fori_loop(0, _CHUNK // 16, grp, 0)

        def round_body(b, carry):
            blk = 2 * b + c

            def load_src(j, a):
                off = (blk * _NS + s) * (_CPT * _CHUNK) + j * _CHUNK
                pltpu.async_copy(srcg_h.at[pl.ds(off, _CHUNK)], srcv[a],
                                 isem[a])

            def load_dst(j, a):
                off = s * (_CPT * _CHUNK) + j * _CHUNK
                pltpu.async_copy(dstp_h.at[pl.ds(off, _CHUNK)], dstv[a],
                                 dsem[a])

            def load_val(j, a):
                off = s * (_CPT * _CHUNK) + j * _CHUNK
                pltpu.async_copy(valp_h.at[pl.ds(off, _CHUNK)], valv[a],
                                 vsem[a])

            def wait_src(a):
                pltpu.make_async_copy(srcg_h.at[pl.ds(0, _CHUNK)], srcv[a],
                                      isem[a]).wait()

            def wait_dst(a):
                pltpu.make_async_copy(dstp_h.at[pl.ds(0, _CHUNK)], dstv[a],
                                      dsem[a]).wait()

            def wait_val(a):
                pltpu.make_async_copy(valp_h.at[pl.ds(0, _CHUNK)], valv[a],
                                      vsem[a]).wait()

            def start_gather(a):
                pltpu.async_copy(tab_h.at[srcv[a]], rows[a], gsem[a])

            def wait_gather(a):
                pltpu.make_async_copy(tab_h.at[srcv[a]], rows[a],
                                      gsem[a]).wait()

            def start_scatter(a):
                pltpu.async_copy(rows[a], acc.at[dstv[a]], ssem[a], add=True)

            def wait_scatter(a):
                pltpu.make_async_copy(rows[a], acc.at[dstv[a]],
                                      ssem[a]).wait()

            def step(j, a, first=False, src_next=True, dst_next=True,
                     gather_next=True):
                nxt = (a + 3) % 4
                wait_gather(a)
                wait_val(a)
                scale(j, a)
                if src_next:
                    load_src(j + 4, a)
                    load_val(j + 4, a)
                if not first:
                    wait_scatter(nxt)
                if dst_next:
                    load_dst(j + 3, nxt)
                if gather_next:
                    wait_src(nxt)
                    start_gather(nxt)
                wait_dst(a)
                start_scatter(a)

            for m in range(4):
                load_src(m, m)
                load_val(m, m)
            for m in range(3):
                load_dst(m, m)
            for m in range(3):
                wait_src(m)
                start_gather(m)
            step(0, 0, first=True)
            step(1, 1)
            step(2, 2)

            def pipe(t, inner):
                j = 4 * t + 3
                step(j, 3)
                step(j + 1, 0)
                step(j + 2, 1)
                step(j + 3, 2)
                return inner
            lax.fori_loop(0, (_CPT - 8) // 4, pipe, 0)

            step(_CPT - 5, 3)
            step(_CPT - 4, 0, src_next=False)
            step(_CPT - 3, 1, src_next=False, dst_next=False,
                 gather_next=False)
            step(_CPT - 2, 2, src_next=False, dst_next=False,
                 gather_next=False)
            step(_CPT - 1, 3, src_next=False, dst_next=False,
                 gather_next=False)
            wait_scatter(3)

            plsc.subcore_barrier()
            pltpu.sync_copy(acc.at[pl.ds(s * _RPT, _RPT)],
                            out_h.at[pl.ds(blk * _NP + s * _RPT, _RPT)])
            zero_acc()
            plsc.subcore_barrier()
            return carry
        lax.fori_loop(0, BATCH, round_body, 0)

    return k(tab, src_g, dst_p, val_p, zrows)


def _cheb_matmul(xt, y1, y2, z, weights, bias):
    def km(x_ref, y1_ref, y2_ref, z_ref, w_ref, b_ref, o_ref):
        a0 = 4.0 * w_ref[0] - 2.0 * w_ref[2]
        a1 = 4.0 * w_ref[1] - 3.0 * w_ref[3]
        a2 = 4.0 * w_ref[2]
        a3 = 4.0 * w_ref[3]
        bb = b_ref[0]
        for b in range(BATCH):
            acc = None
            for t_ref, a in ((x_ref, a0), (y1_ref, a1), (y2_ref, a2), (z_ref, a3)):
                p = (jnp.dot(t_ref[b, 0], a[:_H], preferred_element_type=jnp.float32)
                     + jnp.dot(t_ref[b, 1], a[_H:], preferred_element_type=jnp.float32))
                acc = p if acc is None else acc + p
            o_ref[b] = jnp.maximum(acc + bb[None, :], 0.0)

    tb = pl.BlockSpec((BATCH, 2, _R, _H), lambda i: (0, 0, i, 0))
    return pl.pallas_call(
        km,
        grid=(_NB,),
        in_specs=[tb, tb, tb, tb,
                  pl.BlockSpec((BATCH + 2, IN_CH, OUT_CH), lambda i: (0, 0, 0)),
                  pl.BlockSpec((1, OUT_CH), lambda i: (0, 0))],
        out_specs=pl.BlockSpec((BATCH, _R, OUT_CH), lambda i: (0, i, 0)),
        out_shape=jax.ShapeDtypeStruct((BATCH, N_NODES, OUT_CH), jnp.float32),
    )(xt.reshape(BATCH, 2, _NP, _H),
      y1.reshape(BATCH, 2, _NP, _H),
      y2.reshape(BATCH, 2, _NP, _H),
      z.reshape(BATCH, 2, _NP, _H),
      weights, bias.reshape(1, OUT_CH))


def kernel(inputs, lap_indices, lap_values, weights, bias):
    xp = jnp.pad(inputs, ((0, 0), (0, _NP - N_NODES), (0, 0)))
    xt = (xp.reshape(BATCH, _NP, 2, _H)
          .transpose(0, 2, 1, 3)
          .reshape(2 * BATCH * _NP, _H))
    src = lap_indices[1].astype(jnp.int32)
    dst = lap_indices[0].astype(jnp.int32)
    pad = _EPAD - N_EDGES
    zpad_i = jnp.zeros((pad,), jnp.int32)
    src_p = jnp.concatenate([src, zpad_i])
    dst_p = jnp.concatenate([dst, zpad_i])
    val_p = jnp.concatenate([lap_values.astype(jnp.float32),
                             jnp.zeros((pad,), jnp.float32)])
    offs = (jnp.arange(2 * BATCH, dtype=jnp.int32) * _NP)[:, None]
    src_g = (src_p[None, :] + offs).reshape(-1)

    zrows = jnp.zeros((_RPT, _H), jnp.float32)
    y1 = _spmm_tables(xt, src_g, dst_p, val_p, zrows)
    y2 = _spmm_tables(y1, src_g, dst_p, val_p, zrows)
    z = _spmm_tables(y2, src_g, dst_p, val_p, zrows)
    return _cheb_matmul(xt, y1, y2, z, weights, bias)
